# Initial kernel scaffold; baseline (speedup 1.0000x reference)
import functools

import jax
import jax.numpy as jnp
from jax import lax
from jax.experimental import pallas as pl
from jax.experimental.pallas import tpu as pltpu
from jax.experimental.pallas import tpu_sc as plsc

N = 10000
E = 320000
F = 128
R = 8
NUM_CORES = 2
NUM_SUBCORES = 16
NW = NUM_CORES * NUM_SUBCORES
ROWS_PER_W = 320
NPAD = NW * ROWS_PER_W
SCAN_CHUNK = 8000
MATCH_CAP = 12288
GRP = 128
MATCH_BUF = MATCH_CAP + GRP
IDX_MASK = (1 << 17) - 1


def _pack_body(src_ref, dst_ref, et_ref, w0_ref):
    dst = dst_ref[...]
    src = src_ref[...]
    et = et_ref[...]
    owner = (dst * 52429) >> 24
    dl = dst - owner * 320
    w0_ref[...] = owner * (1 << 26) + dl * (1 << 17) + src * 8 + et


def _pack_edges(src, dst, et):
    s2 = src.reshape(640, 500)
    d2 = dst.reshape(640, 500)
    e2 = et.reshape(640, 500)
    blk = pl.BlockSpec((128, 500), lambda i: (i, 0))
    w0 = pl.pallas_call(
        _pack_body,
        grid=(5,),
        in_specs=[blk, blk, blk],
        out_specs=blk,
        out_shape=jax.ShapeDtypeStruct((640, 500), jnp.int32),
    )(s2, d2, e2)
    return w0.reshape(E)


def _mm_body(x_ref, w_ref, o_ref):
    o_ref[...] = jnp.dot(x_ref[...], w_ref[...],
                         preferred_element_type=jnp.float32)


def _build_table(xp, wcat):
    out = pl.pallas_call(
        _mm_body,
        grid=(NPAD // 512,),
        in_specs=[
            pl.BlockSpec((512, F), lambda i: (i, 0)),
            pl.BlockSpec((F, R * F), lambda i: (0, 0)),
        ],
        out_specs=pl.BlockSpec((512, R * F), lambda i: (i, 0)),
        out_shape=jax.ShapeDtypeStruct((NPAD, R * F), jnp.float32),
    )(xp, wcat)
    return out.reshape(NPAD * R, F)


_mesh = plsc.VectorSubcoreMesh(core_axis_name="c", subcore_axis_name="s")


@functools.partial(
    pl.kernel,
    mesh=_mesh,
    out_type=jax.ShapeDtypeStruct((NPAD * F,), jnp.float32),
    scratch_types=[
        pltpu.VMEM((ROWS_PER_W * F,), jnp.float32),
        pltpu.VMEM((SCAN_CHUNK,), jnp.int32),
        pltpu.VMEM((SCAN_CHUNK,), jnp.float32),
        pltpu.VMEM((MATCH_BUF,), jnp.int32),
        pltpu.VMEM((MATCH_BUF,), jnp.float32),
        pltpu.VMEM((GRP,), jnp.int32),
        pltpu.VMEM((GRP, F), jnp.float32),
        pltpu.SemaphoreType.DMA,
    ],
)
def _sc_scatter(table_hbm, w0_hbm, norm_hbm, out_hbm,
                acc_v, w0_v, nrm_v, mw0_v, mnrm_v, gidx_v, rows_v, sem):
    wid = lax.axis_index("s") * NUM_CORES + lax.axis_index("c")
    zeros_f = jnp.zeros((16,), jnp.float32)
    zeros_i = jnp.zeros((16,), jnp.int32)

    def zero_body(i, carry):
        acc_v[pl.ds(i * 16, 16)] = zeros_f
        return carry

    lax.fori_loop(0, ROWS_PER_W * F // 16, zero_body, 0)

    def chunk_body(c, mcount):
        pltpu.sync_copy(w0_hbm.at[pl.ds(c * SCAN_CHUNK, SCAN_CHUNK)], w0_v)
        pltpu.sync_copy(norm_hbm.at[pl.ds(c * SCAN_CHUNK, SCAN_CHUNK)], nrm_v)

        def scan_body(j, mc):
            w0 = w0_v[pl.ds(j * 16, 16)]
            m = (w0 >> 26) == wid
            cnt = jnp.sum(m.astype(jnp.int32))
            off = jnp.minimum(mc, MATCH_CAP)
            plsc.store_compressed(mw0_v.at[pl.ds(off, 16)], w0, m)
            plsc.store_compressed(mnrm_v.at[pl.ds(off, 16)],
                                  nrm_v[pl.ds(j * 16, 16)], m)
            return mc + cnt

        return lax.fori_loop(0, SCAN_CHUNK // 16, scan_body, mcount)

    mcount = lax.fori_loop(0, E // SCAN_CHUNK, chunk_body, jnp.int32(0))
    mcount = jnp.minimum(mcount, MATCH_CAP)

    def pad_body(t, carry):
        mw0_v[pl.ds(mcount + t * 16, 16)] = zeros_i
        mnrm_v[pl.ds(mcount + t * 16, 16)] = zeros_f
        return carry

    lax.fori_loop(0, GRP // 16, pad_body, 0)
    ngroups = (mcount + GRP - 1) // GRP

    def grp_body(g, carry):
        base = g * GRP

        def dec_body(t, c2):
            gidx_v[pl.ds(t * 16, 16)] = (
                mw0_v[pl.ds(base + t * 16, 16)] & IDX_MASK)
            return c2

        lax.fori_loop(0, GRP // 16, dec_body, 0)
        pltpu.async_copy(table_hbm.at[gidx_v], rows_v, sem).wait()

        def edge_body(j, c2):
            w0s = mw0_v[base + j]
            nrm = mnrm_v[base + j]
            ob = ((w0s >> 17) & 0x1FF) * F
            nv = jnp.full((16,), nrm)
            for k in range(F // 16):
                r = rows_v[j, pl.ds(k * 16, 16)]
                plsc.addupdate(acc_v.at[pl.ds(ob + k * 16, 16)], r * nv)
            return c2

        lax.fori_loop(0, GRP, edge_body, 0)
        return carry

    lax.fori_loop(0, ngroups, grp_body, 0)

    pltpu.sync_copy(
        acc_v, out_hbm.at[pl.ds(wid * (ROWS_PER_W * F), ROWS_PER_W * F)])


def kernel(x, edge_index, edge_type, edge_norm, weight):
    src = edge_index[0].astype(jnp.int32)
    dst = edge_index[1].astype(jnp.int32)
    w0 = _pack_edges(src, dst, edge_type.astype(jnp.int32))

    w4 = weight.reshape(R, 8, 16, 16)
    eye = jnp.eye(8, dtype=x.dtype)
    wcat = (w4.transpose(1, 2, 0, 3)[:, :, :, None, :]
            * eye[:, None, None, :, None]).reshape(F, R * F)

    xp = jnp.pad(x, ((0, NPAD - N), (0, 0)))
    table = _build_table(xp, wcat)

    out_flat = _sc_scatter(table, w0, edge_norm)
    return out_flat.reshape(NPAD, F)[:N]


# speedup vs baseline: 12.5576x; 12.5576x over previous
import functools

import jax
import jax.numpy as jnp
from jax import lax
from jax.experimental import pallas as pl
from jax.experimental.pallas import tpu as pltpu
from jax.experimental.pallas import tpu_sc as plsc

N = 10000
E = 320000
F = 128
R = 8
NUM_CORES = 2
NUM_SUBCORES = 16
NW = NUM_CORES * NUM_SUBCORES
NPAD = 10240
CHUNK = 128
E_PER_W = 10112
E_PAD = NW * E_PER_W
N_CHUNKS = E_PER_W // CHUNK
IDX_MASK = (1 << 17) - 1
ROWS_PER_TILE = NPAD // NUM_SUBCORES


def _pack_body(src_ref, dst_ref, et_ref, w0_ref):
    w0_ref[...] = (dst_ref[...] * (1 << 17)
                   + src_ref[...] * 8 + et_ref[...])


def _pack_edges(src, dst, et):
    s2 = src.reshape(640, 500)
    d2 = dst.reshape(640, 500)
    e2 = et.reshape(640, 500)
    blk = pl.BlockSpec((128, 500), lambda i: (i, 0))
    w0 = pl.pallas_call(
        _pack_body,
        grid=(5,),
        in_specs=[blk, blk, blk],
        out_specs=blk,
        out_shape=jax.ShapeDtypeStruct((640, 500), jnp.int32),
    )(s2, d2, e2)
    return w0.reshape(E)


def _mm_body(x_ref, w_ref, o_ref):
    o_ref[...] = jnp.dot(x_ref[...], w_ref[...],
                         preferred_element_type=jnp.float32)


def _build_table(xp, wcat):
    out = pl.pallas_call(
        _mm_body,
        grid=(NPAD // 512,),
        in_specs=[
            pl.BlockSpec((512, F), lambda i: (i, 0)),
            pl.BlockSpec((F, R * F), lambda i: (0, 0)),
        ],
        out_specs=pl.BlockSpec((512, R * F), lambda i: (i, 0)),
        out_shape=jax.ShapeDtypeStruct((NPAD, R * F), jnp.float32),
    )(xp, wcat)
    return out.reshape(NPAD * R, F)


def _sum_body(a_ref, b_ref, o_ref):
    o_ref[...] = a_ref[...] + b_ref[...]


def _sum_parts(p0, p1):
    blk = pl.BlockSpec((512, F), lambda i: (i, 0))
    return pl.pallas_call(
        _sum_body,
        grid=(NPAD // 512,),
        in_specs=[blk, blk],
        out_specs=blk,
        out_shape=jax.ShapeDtypeStruct((NPAD, F), jnp.float32),
    )(p0, p1)


_mesh = plsc.VectorSubcoreMesh(core_axis_name="c", subcore_axis_name="s")


@functools.partial(
    pl.kernel,
    mesh=_mesh,
    out_type=jax.ShapeDtypeStruct((2 * NPAD, F), jnp.float32),
    scratch_types=[
        pltpu.VMEM((CHUNK,), jnp.int32),
        pltpu.VMEM((CHUNK,), jnp.float32),
        pltpu.VMEM((CHUNK,), jnp.int32),
        pltpu.VMEM((CHUNK,), jnp.int32),
        pltpu.VMEM((CHUNK, F), jnp.float32),
        pltpu.VMEM_SHARED((NPAD, F), jnp.float32),
        pltpu.SemaphoreType.DMA,
    ],
)
def _sc_scatter(table_hbm, w0_hbm, norm_hbm, out_hbm,
                w0_v, nrm_v, gidx_v, dsti_v, rows_v, acc_sh, sem):
    cid = lax.axis_index("c")
    sid = lax.axis_index("s")
    wid = sid * NUM_CORES + cid
    zeros_f = jnp.zeros((16,), jnp.float32)

    def zrow_body(i, carry):
        def zcol_body(k, c2):
            rows_v[i, pl.ds(k * 16, 16)] = zeros_f
            return c2
        lax.fori_loop(0, F // 16, zcol_body, 0)
        return carry

    lax.fori_loop(0, CHUNK, zrow_body, 0)

    def zdma_body(p, carry):
        pltpu.sync_copy(
            rows_v,
            acc_sh.at[pl.ds(sid * ROWS_PER_TILE + p * CHUNK, CHUNK)])
        return carry

    lax.fori_loop(0, ROWS_PER_TILE // CHUNK, zdma_body, 0)
    plsc.subcore_barrier()

    def chunk_body(c, carry):
        off = wid * E_PER_W + c * CHUNK
        pltpu.sync_copy(w0_hbm.at[pl.ds(off, CHUNK)], w0_v)
        pltpu.sync_copy(norm_hbm.at[pl.ds(off, CHUNK)], nrm_v)

        def dec_body(t, c2):
            w = w0_v[pl.ds(t * 16, 16)]
            gidx_v[pl.ds(t * 16, 16)] = w & IDX_MASK
            dsti_v[pl.ds(t * 16, 16)] = w >> 17
            return c2

        lax.fori_loop(0, CHUNK // 16, dec_body, 0)
        pltpu.async_copy(table_hbm.at[gidx_v], rows_v, sem).wait()

        def scale_body(t, c2):
            vn = nrm_v[pl.ds(t * 16, 16)]
            for j2 in range(16):
                nv = jnp.full((16,), vn[j2])
                rr = t * 16 + j2
                for k in range(F // 16):
                    rows_v[rr, pl.ds(k * 16, 16)] = (
                        rows_v[rr, pl.ds(k * 16, 16)] * nv)
            return c2

        lax.fori_loop(0, CHUNK // 16, scale_body, 0)
        pltpu.sync_copy(rows_v, acc_sh.at[dsti_v], add=True)
        return carry

    lax.fori_loop(0, N_CHUNKS, chunk_body, 0)
    plsc.subcore_barrier()

    def drain_body(p, carry):
        row0 = sid * ROWS_PER_TILE + p * CHUNK
        pltpu.sync_copy(acc_sh.at[pl.ds(row0, CHUNK)],
                        out_hbm.at[pl.ds(cid * NPAD + row0, CHUNK)])
        return carry

    lax.fori_loop(0, ROWS_PER_TILE // CHUNK, drain_body, 0)


def kernel(x, edge_index, edge_type, edge_norm, weight):
    src = edge_index[0].astype(jnp.int32)
    dst = edge_index[1].astype(jnp.int32)
    w0 = _pack_edges(src, dst, edge_type.astype(jnp.int32))
    w0p = jnp.pad(w0, (0, E_PAD - E))
    normp = jnp.pad(edge_norm, (0, E_PAD - E))

    w4 = weight.reshape(R, 8, 16, 16)
    eye = jnp.eye(8, dtype=x.dtype)
    wcat = (w4.transpose(1, 2, 0, 3)[:, :, :, None, :]
            * eye[:, None, None, :, None]).reshape(F, R * F)

    xp = jnp.pad(x, ((0, NPAD - N), (0, 0)))
    table = _build_table(xp, wcat)

    parts = _sc_scatter(table, w0p, normp)
    out = _sum_parts(parts[:NPAD], parts[NPAD:])
    return out[:N]
